# baseline (speedup 1.0000x reference)
import functools

import jax
import jax.numpy as jnp
from jax import lax
from jax.experimental import pallas as pl
from jax.experimental.pallas import tpu as pltpu
from jax.experimental.pallas import tpu_sc as plsc

NUM_CORES = 2
NUM_SUBCORES = 16
NUM_WORKERS = NUM_CORES * NUM_SUBCORES
LANES = 16

B = 16384
D = 64
BPW = B // NUM_WORKERS
HALF = BPW // 2
HGROUPS = HALF // LANES
FETCHROWS = 128


def _fetch_window(ent_ref, relemb_ref, hidx, ridx, tidx, hbuf, rbuf, tbuf,
                  sem, r0):
    def fetch(gr, carry):
        copies = []
        for k in range(FETCHROWS // LANES):
            s0 = gr * FETCHROWS + k * LANES
            hv = hidx[pl.ds(r0 + s0, LANES)]
            rv = ridx[pl.ds(r0 + s0, LANES)]
            tv = tidx[pl.ds(r0 + s0, LANES)]
            for j in range(LANES):
                q = pl.ds(s0 + j, 1)
                copies.append(pltpu.async_copy(
                    ent_ref.at[pl.ds(hv[j], 1)], hbuf.at[q], sem))
                copies.append(pltpu.async_copy(
                    relemb_ref.at[pl.ds(rv[j], 1)], rbuf.at[q], sem))
                copies.append(pltpu.async_copy(
                    ent_ref.at[pl.ds(tv[j], 1)], tbuf.at[q], sem))
        for c in copies:
            c.wait()
        return carry

    lax.fori_loop(0, HALF // FETCHROWS, fetch, 0)


def _compute_window(hbuf, rbuf, tbuf, score, g0):
    lane = lax.iota(jnp.int32, LANES)

    def group(g, carry):
        rows = g * LANES + lane
        acc = jnp.zeros((LANES,), jnp.float32)
        for d in range(D):
            col = jnp.full((LANES,), d, jnp.int32)
            hv = plsc.load_gather(hbuf, [rows, col])
            rv = plsc.load_gather(rbuf, [rows, col])
            tv = plsc.load_gather(tbuf, [rows, col])
            acc = acc + hv * rv * tv
        score[pl.ds((g0 + g) * LANES, LANES)] = acc
        return carry

    lax.fori_loop(0, HGROUPS, group, 0)


def _body(head_ref, rel_ref, tail_ref, ent_ref, relemb_ref, out_ref,
          hidx, ridx, tidx, hbuf, rbuf, tbuf, score, sem):
    wid = lax.axis_index("s") * NUM_CORES + lax.axis_index("c")
    base = wid * BPW

    pltpu.sync_copy(head_ref.at[pl.ds(base, BPW)], hidx)
    pltpu.sync_copy(rel_ref.at[pl.ds(base, BPW)], ridx)
    pltpu.sync_copy(tail_ref.at[pl.ds(base, BPW)], tidx)

    for half in range(2):
        _fetch_window(ent_ref, relemb_ref, hidx, ridx, tidx,
                      hbuf, rbuf, tbuf, sem, half * HALF)
        _compute_window(hbuf, rbuf, tbuf, score, half * HGROUPS)

    pltpu.sync_copy(score, out_ref.at[pl.ds(base, BPW)])


@functools.cache
def _build():
    return pl.kernel(
        _body,
        out_type=jax.ShapeDtypeStruct((B,), jnp.float32),
        mesh=plsc.VectorSubcoreMesh(core_axis_name="c", subcore_axis_name="s"),
        compiler_params=pltpu.CompilerParams(needs_layout_passes=False),
        scratch_types=[
            pltpu.VMEM((BPW,), jnp.int32),
            pltpu.VMEM((BPW,), jnp.int32),
            pltpu.VMEM((BPW,), jnp.int32),
            pltpu.VMEM((HALF, D), jnp.float32),
            pltpu.VMEM((HALF, D), jnp.float32),
            pltpu.VMEM((HALF, D), jnp.float32),
            pltpu.VMEM((BPW,), jnp.float32),
            pltpu.SemaphoreType.DMA,
        ],
    )


def kernel(head, relation, tail, entity_embeddings, relation_embeddings):
    return _build()(head, relation, tail,
                    entity_embeddings, relation_embeddings)


# speedup vs baseline: 1.0104x; 1.0104x over previous
import functools

import jax
import jax.numpy as jnp
from jax import lax
from jax.experimental import pallas as pl
from jax.experimental.pallas import tpu as pltpu
from jax.experimental.pallas import tpu_sc as plsc

NUM_CORES = 2
NUM_SUBCORES = 16
NUM_WORKERS = NUM_CORES * NUM_SUBCORES
LANES = 16

B = 16384
D = 64
BPW = B // NUM_WORKERS
HALF = BPW // 2
HGROUPS = HALF // LANES
FETCHROWS = 64


def _fetch_window(ent_ref, relemb_ref, hidx, ridx, tidx, hbuf, rbuf, tbuf,
                  sem, r0):
    def fetch(gr, carry):
        copies = []
        for k in range(FETCHROWS // LANES):
            s0 = gr * FETCHROWS + k * LANES
            hv = hidx[pl.ds(r0 + s0, LANES)]
            rv = ridx[pl.ds(r0 + s0, LANES)]
            tv = tidx[pl.ds(r0 + s0, LANES)]
            for j in range(LANES):
                q = pl.ds(s0 + j, 1)
                copies.append(pltpu.async_copy(
                    ent_ref.at[pl.ds(hv[j], 1)], hbuf.at[q], sem))
                copies.append(pltpu.async_copy(
                    relemb_ref.at[pl.ds(rv[j], 1)], rbuf.at[q], sem))
                copies.append(pltpu.async_copy(
                    ent_ref.at[pl.ds(tv[j], 1)], tbuf.at[q], sem))
        for c in copies:
            c.wait()
        return carry

    lax.fori_loop(0, HALF // FETCHROWS, fetch, 0)


def _compute_window(hbuf, rbuf, tbuf, score, g0):
    lane = lax.iota(jnp.int32, LANES)

    def group(g, carry):
        rows = g * LANES + lane
        acc = jnp.zeros((LANES,), jnp.float32)
        for d in range(D):
            col = jnp.full((LANES,), d, jnp.int32)
            hv = plsc.load_gather(hbuf, [rows, col])
            rv = plsc.load_gather(rbuf, [rows, col])
            tv = plsc.load_gather(tbuf, [rows, col])
            acc = acc + hv * rv * tv
        score[pl.ds((g0 + g) * LANES, LANES)] = acc
        return carry

    lax.fori_loop(0, HGROUPS, group, 0)


def _body(head_ref, rel_ref, tail_ref, ent_ref, relemb_ref, out_ref,
          hidx, ridx, tidx, hbuf, rbuf, tbuf, score, sem):
    wid = lax.axis_index("s") * NUM_CORES + lax.axis_index("c")
    base = wid * BPW

    pltpu.sync_copy(head_ref.at[pl.ds(base, BPW)], hidx)
    pltpu.sync_copy(rel_ref.at[pl.ds(base, BPW)], ridx)
    pltpu.sync_copy(tail_ref.at[pl.ds(base, BPW)], tidx)

    for half in range(2):
        _fetch_window(ent_ref, relemb_ref, hidx, ridx, tidx,
                      hbuf, rbuf, tbuf, sem, half * HALF)
        _compute_window(hbuf, rbuf, tbuf, score, half * HGROUPS)

    pltpu.sync_copy(score, out_ref.at[pl.ds(base, BPW)])


@functools.cache
def _build():
    return pl.kernel(
        _body,
        out_type=jax.ShapeDtypeStruct((B,), jnp.float32),
        mesh=plsc.VectorSubcoreMesh(core_axis_name="c", subcore_axis_name="s"),
        compiler_params=pltpu.CompilerParams(needs_layout_passes=False),
        scratch_types=[
            pltpu.VMEM((BPW,), jnp.int32),
            pltpu.VMEM((BPW,), jnp.int32),
            pltpu.VMEM((BPW,), jnp.int32),
            pltpu.VMEM((HALF, D), jnp.float32),
            pltpu.VMEM((HALF, D), jnp.float32),
            pltpu.VMEM((HALF, D), jnp.float32),
            pltpu.VMEM((BPW,), jnp.float32),
            pltpu.SemaphoreType.DMA,
        ],
    )


def kernel(head, relation, tail, entity_embeddings, relation_embeddings):
    return _build()(head, relation, tail,
                    entity_embeddings, relation_embeddings)
